# Initial kernel scaffold; baseline (speedup 1.0000x reference)
import jax
import jax.numpy as jnp
from jax.experimental import pallas as pl


def kernel(notes, onsets, durations, x_lengths, emb_table):
    raise NotImplementedError("write your pallas kernel here")


# speedup vs baseline: 2.6643x; 2.6643x over previous
import functools
import jax
import jax.numpy as jnp
from jax import lax
from jax.experimental import pallas as pl
from jax.experimental.pallas import tpu as pltpu
from jax.experimental.pallas import tpu_sc as plsc

B, T = 4096, 200
VOCAB, DIM = 127, 64
N = B * T

_info = plsc.get_sparse_core_info()
NC, NS, L = _info.num_cores, _info.num_subcores, _info.num_lanes
NW = NC * NS

CHUNK = 128
PER_W = N // NW
NCHUNK = PER_W // CHUNK


def _emb_kernel(notes_hbm, table_hbm, out_hbm, notes_v, idx_v, rows_v, sem):
    wid = lax.axis_index("s") * NC + lax.axis_index("c")
    w_base = wid * PER_W

    def body(c, carry):
        base = w_base + c * CHUNK
        pltpu.sync_copy(notes_hbm.at[pl.ds(base, CHUNK)], notes_v)
        for i in range(CHUNK // L):
            sl = pl.ds(i * L, L)
            idx_v[sl] = (notes_v[sl] * 127.0).astype(jnp.int32)
        pltpu.async_copy(table_hbm.at[idx_v], rows_v, sem).wait()
        pltpu.sync_copy(rows_v, out_hbm.at[pl.ds(base, CHUNK)])
        return carry

    lax.fori_loop(0, NCHUNK, body, 0)


@jax.jit
def _emb_lookup(notes_flat, emb_table):
    mesh = plsc.VectorSubcoreMesh(core_axis_name="c", subcore_axis_name="s")
    return pl.kernel(
        _emb_kernel,
        out_type=jax.ShapeDtypeStruct((N, DIM), jnp.float32),
        mesh=mesh,
        scratch_types=[
            pltpu.VMEM((CHUNK,), jnp.float32),
            pltpu.VMEM((CHUNK,), jnp.int32),
            pltpu.VMEM((CHUNK, DIM), jnp.float32),
            pltpu.SemaphoreType.DMA,
        ],
        compiler_params=pltpu.CompilerParams(use_tc_tiling_on_sc=False),
    )(notes_flat, emb_table)


def kernel(notes, onsets, durations, x_lengths, emb_table):
    notes_flat = notes.reshape(N)
    out = _emb_lookup(notes_flat, emb_table)
    return out.reshape(B, T, DIM)


# speedup vs baseline: 2.6869x; 1.0085x over previous
import jax
import jax.numpy as jnp
from jax import lax
from jax.experimental import pallas as pl
from jax.experimental.pallas import tpu as pltpu
from jax.experimental.pallas import tpu_sc as plsc

B, T = 4096, 200
VOCAB, DIM = 127, 64
N = B * T

_info = plsc.get_sparse_core_info()
NC, NS, L = _info.num_cores, _info.num_subcores, _info.num_lanes
NW = NC * NS

CHUNK = 128
GPB = 4
SUPER = CHUNK * GPB
PER_W = N // NW
NSUPER = PER_W // SUPER


def _emb_kernel(notes_hbm, table_hbm, out_hbm, notes_all, idx_all, rows, gsem, osem):
    wid = lax.axis_index("s") * NC + lax.axis_index("c")
    w_base = wid * PER_W

    pltpu.sync_copy(notes_hbm.at[pl.ds(w_base, PER_W)], notes_all)

    def conv(c, carry):
        for i in range(CHUNK // L):
            sl = pl.ds(c * CHUNK + i * L, L)
            idx_all[sl] = (notes_all[sl] * 127.0).astype(jnp.int32)
        return carry

    lax.fori_loop(0, PER_W // CHUNK, conv, 0)

    def body(g, carry):
        base = w_base + g * SUPER
        boff = lax.rem(g, 2) * SUPER

        @pl.when(g >= 2)
        def _drain():
            pltpu.make_async_copy(
                rows.at[pl.ds(boff, SUPER)],
                out_hbm.at[pl.ds(base, SUPER)],
                osem,
            ).wait()

        copies = []
        for j in range(GPB):
            idx_sl = idx_all.at[pl.ds(g * SUPER + j * CHUNK, CHUNK)]
            copies.append(
                pltpu.async_copy(
                    table_hbm.at[idx_sl],
                    rows.at[pl.ds(boff + j * CHUNK, CHUNK)],
                    gsem,
                )
            )
        for cp in copies:
            cp.wait()

        pltpu.async_copy(
            rows.at[pl.ds(boff, SUPER)],
            out_hbm.at[pl.ds(base, SUPER)],
            osem,
        )
        return carry

    lax.fori_loop(0, NSUPER, body, 0)

    for k in range(2):
        pltpu.make_async_copy(
            rows.at[pl.ds(0, SUPER)],
            out_hbm.at[pl.ds(w_base, SUPER)],
            osem,
        ).wait()


@jax.jit
def _emb_lookup(notes_flat, emb_table):
    mesh = plsc.VectorSubcoreMesh(core_axis_name="c", subcore_axis_name="s")
    return pl.kernel(
        _emb_kernel,
        out_type=jax.ShapeDtypeStruct((N, DIM), jnp.float32),
        mesh=mesh,
        scratch_types=[
            pltpu.VMEM((PER_W,), jnp.float32),
            pltpu.VMEM((PER_W,), jnp.int32),
            pltpu.VMEM((2 * SUPER, DIM), jnp.float32),
            pltpu.SemaphoreType.DMA,
            pltpu.SemaphoreType.DMA,
        ],
        compiler_params=pltpu.CompilerParams(use_tc_tiling_on_sc=False),
    )(notes_flat, emb_table)


def kernel(notes, onsets, durations, x_lengths, emb_table):
    out = _emb_lookup(notes.reshape(N), emb_table)
    return out.reshape(B, T, DIM)


# speedup vs baseline: 4.9897x; 1.8571x over previous
import jax
import jax.numpy as jnp
from jax import lax
from jax.experimental import pallas as pl
from jax.experimental.pallas import tpu as pltpu
from jax.experimental.pallas import tpu_sc as plsc

B, T = 4096, 200
VOCAB, DIM = 127, 64
N = B * T

_info = plsc.get_sparse_core_info()
NC, NS, L = _info.num_cores, _info.num_subcores, _info.num_lanes
NW = NC * NS

CHUNK = 128
GPB = 4
SUPER = CHUNK * GPB
PER_W = N // NW
NSUPER = PER_W // SUPER


def _emb_kernel(notes_hbm, table_hbm, out_hbm, table_v, notes_all, idx_all, rows, gsem, osem):
    wid = lax.axis_index("s") * NC + lax.axis_index("c")
    w_base = wid * PER_W

    @pl.when(lax.axis_index("s") == 0)
    def _stage_table():
        pltpu.sync_copy(table_hbm, table_v)

    pltpu.sync_copy(notes_hbm.at[pl.ds(w_base, PER_W)], notes_all)
    plsc.subcore_barrier()

    def conv(c, carry):
        for i in range(CHUNK // L):
            sl = pl.ds(c * CHUNK + i * L, L)
            idx_all[sl] = (notes_all[sl] * 127.0).astype(jnp.int32)
        return carry

    lax.fori_loop(0, PER_W // CHUNK, conv, 0)

    def body(g, carry):
        base = w_base + g * SUPER
        boff = lax.rem(g, 2) * SUPER

        @pl.when(g >= 2)
        def _drain():
            pltpu.make_async_copy(
                rows.at[pl.ds(boff, SUPER)],
                out_hbm.at[pl.ds(base, SUPER)],
                osem,
            ).wait()

        copies = []
        for j in range(GPB):
            idx_sl = idx_all.at[pl.ds(g * SUPER + j * CHUNK, CHUNK)]
            copies.append(
                pltpu.async_copy(
                    table_v.at[idx_sl],
                    rows.at[pl.ds(boff + j * CHUNK, CHUNK)],
                    gsem,
                )
            )
        for cp in copies:
            cp.wait()

        pltpu.async_copy(
            rows.at[pl.ds(boff, SUPER)],
            out_hbm.at[pl.ds(base, SUPER)],
            osem,
        )
        return carry

    lax.fori_loop(0, NSUPER, body, 0)

    for k in range(2):
        pltpu.make_async_copy(
            rows.at[pl.ds(0, SUPER)],
            out_hbm.at[pl.ds(w_base, SUPER)],
            osem,
        ).wait()


@jax.jit
def _emb_lookup(notes_flat, emb_table):
    mesh = plsc.VectorSubcoreMesh(core_axis_name="c", subcore_axis_name="s")
    return pl.kernel(
        _emb_kernel,
        out_type=jax.ShapeDtypeStruct((N, DIM), jnp.float32),
        mesh=mesh,
        scratch_types=[
            pltpu.VMEM_SHARED((VOCAB, DIM), jnp.float32),
            pltpu.VMEM((PER_W,), jnp.float32),
            pltpu.VMEM((PER_W,), jnp.int32),
            pltpu.VMEM((2 * SUPER, DIM), jnp.float32),
            pltpu.SemaphoreType.DMA,
            pltpu.SemaphoreType.DMA,
        ],
        compiler_params=pltpu.CompilerParams(use_tc_tiling_on_sc=False),
    )(notes_flat, emb_table)


def kernel(notes, onsets, durations, x_lengths, emb_table):
    out = _emb_lookup(notes.reshape(N), emb_table)
    return out.reshape(B, T, DIM)


# speedup vs baseline: 6.7105x; 1.3449x over previous
import jax
import jax.numpy as jnp
from jax import lax
from jax.experimental import pallas as pl
from jax.experimental.pallas import tpu as pltpu
from jax.experimental.pallas import tpu_sc as plsc

B, T = 4096, 200
VOCAB, DIM = 127, 64
N = B * T

_info = plsc.get_sparse_core_info()
NC, NS, L = _info.num_cores, _info.num_subcores, _info.num_lanes
NW = NC * NS

CHUNK = 128
GPB = 4
SUPER = CHUNK * GPB
PER_W = N // NW
NSUPER = PER_W // SUPER


def _emb_kernel(notes_hbm, table_hbm, out_hbm, table_v, notes_all, idx_all, rows, gsem, osem):
    wid = lax.axis_index("s") * NC + lax.axis_index("c")
    w_base = wid * PER_W

    @pl.when(lax.axis_index("s") == 0)
    def _stage_table():
        pltpu.sync_copy(table_hbm, table_v)

    pltpu.sync_copy(notes_hbm.at[pl.ds(w_base, PER_W)], notes_all)
    plsc.subcore_barrier()

    def conv(c, carry):
        for i in range(CHUNK // L):
            sl = pl.ds(c * CHUNK + i * L, L)
            idx_all[sl] = (notes_all[sl] * 127.0).astype(jnp.int32)
        return carry

    lax.fori_loop(0, PER_W // CHUNK, conv, 0)

    def body(g, carry):
        base = w_base + g * SUPER
        boff = lax.rem(g, 2) * SUPER

        @pl.when(g >= 2)
        def _drain():
            pltpu.make_async_copy(
                rows.at[pl.ds(boff, SUPER)],
                out_hbm.at[pl.ds(base, SUPER)],
                osem,
            ).wait()

        copies = []
        for j in range(GPB):
            idx_sl = idx_all.at[pl.ds(g * SUPER + j * CHUNK, CHUNK)]
            copies.append(
                pltpu.async_copy(
                    table_v.at[idx_sl],
                    rows.at[pl.ds(boff + j * CHUNK, CHUNK)],
                    gsem,
                )
            )
        for cp in copies:
            cp.wait()

        pltpu.async_copy(
            rows.at[pl.ds(boff, SUPER)],
            out_hbm.at[pl.ds(base, SUPER)],
            osem,
        )
        return carry

    lax.fori_loop(0, NSUPER, body, 0)

    for k in range(2):
        pltpu.make_async_copy(
            rows.at[pl.ds(0, SUPER)],
            out_hbm.at[pl.ds(w_base, SUPER)],
            osem,
        ).wait()


@jax.jit
def _emb_lookup(notes_flat, emb_table):
    mesh = plsc.VectorSubcoreMesh(core_axis_name="c", subcore_axis_name="s")
    return pl.kernel(
        _emb_kernel,
        out_type=jax.ShapeDtypeStruct((N, DIM), jnp.float32),
        mesh=mesh,
        scratch_types=[
            pltpu.VMEM_SHARED((VOCAB, DIM), jnp.float32),
            pltpu.VMEM((PER_W,), jnp.float32),
            pltpu.VMEM((PER_W,), jnp.int32),
            pltpu.VMEM((2 * SUPER, DIM), jnp.float32),
            pltpu.SemaphoreType.DMA,
            pltpu.SemaphoreType.DMA,
        ],
        compiler_params=pltpu.CompilerParams(use_tc_tiling_on_sc=False),
    )(notes_flat, emb_table)


TD = T * DIM


def _tr_kernel(x_ref, o_ref):
    o_ref[...] = x_ref[...].T


@jax.jit
def _to_device_layout(y):
    return pl.pallas_call(
        _tr_kernel,
        grid=(B // 128,),
        in_specs=[pl.BlockSpec((128, TD), lambda i: (i, 0))],
        out_specs=pl.BlockSpec((TD, 128), lambda i: (0, i)),
        out_shape=jax.ShapeDtypeStruct((TD, B), jnp.float32),
    )(y)


def kernel(notes, onsets, durations, x_lengths, emb_table):
    out = _emb_lookup(notes.reshape(N), emb_table)
    outT = _to_device_layout(out.reshape(B, TD))
    return jnp.transpose(outT.reshape(T, DIM, B), (2, 0, 1))


# speedup vs baseline: 11.8133x; 1.7604x over previous
import jax
import jax.numpy as jnp
from jax import lax
from jax.experimental import pallas as pl
from jax.experimental.pallas import tpu as pltpu
from jax.experimental.pallas import tpu_sc as plsc

B, T = 4096, 200
VOCAB, DIM = 127, 64
N = B * T

_info = plsc.get_sparse_core_info()
NC, NS, L = _info.num_cores, _info.num_subcores, _info.num_lanes
NW = NC * NS

CHUNK = 128
GPB = 2
SUPER = CHUNK * GPB
NBUF = 4
PER_W = N // NW
NSUPER = PER_W // SUPER


def _emb_kernel(notes_hbm, table_hbm, out_hbm, table_v, notes_all, idx_all, rows, g0, g1, g2, g3, osem):
    gsems = (g0, g1, g2, g3)
    wid = lax.axis_index("s") * NC + lax.axis_index("c")
    w_base = wid * PER_W

    @pl.when(lax.axis_index("s") == 0)
    def _stage_table():
        pltpu.sync_copy(table_hbm, table_v)

    pltpu.sync_copy(notes_hbm.at[pl.ds(w_base, PER_W)], notes_all)
    plsc.subcore_barrier()

    def conv(c, carry):
        for i in range(CHUNK // L):
            sl = pl.ds(c * CHUNK + i * L, L)
            idx_all[sl] = (notes_all[sl] * 127.0).astype(jnp.int32)
        return carry

    lax.fori_loop(0, PER_W // CHUNK, conv, 0)

    def fire(g, p):
        for j in range(GPB):
            idx_sl = idx_all.at[pl.ds(g * SUPER + j * CHUNK, CHUNK)]
            pltpu.async_copy(
                table_v.at[idx_sl],
                rows.at[pl.ds(p * SUPER + j * CHUNK, CHUNK)],
                gsems[p],
            )

    def gwait(p):
        for j in range(GPB):
            pltpu.make_async_copy(
                table_v.at[idx_all.at[pl.ds(0, CHUNK)]],
                rows.at[pl.ds(p * SUPER + j * CHUNK, CHUNK)],
                gsems[p],
            ).wait()

    fire(0, 0)
    fire(1, 1)

    def body(gg, carry):
        for p in range(NBUF):
            g = gg * NBUF + p
            gwait(p)
            pltpu.async_copy(
                rows.at[pl.ds(p * SUPER, SUPER)],
                out_hbm.at[pl.ds(w_base + g * SUPER, SUPER)],
                osem,
            )

            @pl.when(g >= 2)
            def _drain():
                pltpu.make_async_copy(
                    rows.at[pl.ds(p * SUPER, SUPER)],
                    out_hbm.at[pl.ds(w_base, SUPER)],
                    osem,
                ).wait()

            @pl.when(g + 2 < NSUPER)
            def _prefetch():
                fire(g + 2, (p + 2) % NBUF)
        return carry

    lax.fori_loop(0, NSUPER // NBUF, body, 0)

    for k in range(2):
        pltpu.make_async_copy(
            rows.at[pl.ds(0, SUPER)],
            out_hbm.at[pl.ds(w_base, SUPER)],
            osem,
        ).wait()


@jax.jit
def _emb_lookup(notes_flat, emb_table):
    mesh = plsc.VectorSubcoreMesh(core_axis_name="c", subcore_axis_name="s")
    return pl.kernel(
        _emb_kernel,
        out_type=jax.ShapeDtypeStruct((N, DIM), jnp.float32),
        mesh=mesh,
        scratch_types=[
            pltpu.VMEM_SHARED((VOCAB, DIM), jnp.float32),
            pltpu.VMEM((PER_W,), jnp.float32),
            pltpu.VMEM((PER_W,), jnp.int32),
            pltpu.VMEM((NBUF * SUPER, DIM), jnp.float32),
            pltpu.SemaphoreType.DMA,
            pltpu.SemaphoreType.DMA,
            pltpu.SemaphoreType.DMA,
            pltpu.SemaphoreType.DMA,
            pltpu.SemaphoreType.DMA,
        ],
        compiler_params=pltpu.CompilerParams(use_tc_tiling_on_sc=False),
    )(notes_flat, emb_table)


TD = T * DIM


def _tr_kernel(x_ref, o_ref):
    x = x_ref[...].reshape(256, TD)
    o_ref[...] = x.T


@jax.jit
def _to_device_layout(y128):
    return pl.pallas_call(
        _tr_kernel,
        grid=(B // 256,),
        in_specs=[pl.BlockSpec((2 * TD, 128), lambda i: (i, 0))],
        out_specs=pl.BlockSpec((TD, 256), lambda i: (0, i)),
        out_shape=jax.ShapeDtypeStruct((TD, B), jnp.float32),
    )(y128)


def kernel(notes, onsets, durations, x_lengths, emb_table):
    out = _emb_lookup(notes.reshape(N), emb_table)
    outT = _to_device_layout(out.reshape(N * DIM // 128, 128))
    return jnp.transpose(outT.reshape(T, DIM, B), (2, 0, 1))
